# baseline (speedup 1.0000x reference)
import functools

import jax
import jax.numpy as jnp
from jax import lax
from jax.experimental import pallas as pl
from jax.experimental.pallas import tpu as pltpu
from jax.experimental.pallas import tpu_sc as plsc

_N = 10000
_E = 320000
_D = 128
_NC = 2
_NS = 16
_NW = _NC * _NS
_EPW = _E // _NW
_K = 64
_NCH = _EPW // _K
_TAIL = _EPW - _NCH * _K
_NZC = -(-_N // _K)
_RCPT = -(-_NZC // _NS)
_NB = 2


def _sc_conv_body(node_hbm, packed_hbm, edge_hbm, out_hbm, *refs):
    packed_all = refs[0]
    srcv = refs[1:1 + _NB]
    dstv = refs[1 + _NB:1 + 2 * _NB]
    srcv_t = refs[1 + 2 * _NB]
    dstv_t = refs[2 + 2 * _NB]
    rows = refs[3 + 2 * _NB:3 + 3 * _NB]
    msg = refs[3 + 3 * _NB:3 + 4 * _NB]
    acc_sh = refs[3 + 4 * _NB]
    sems = refs[4 + 4 * _NB:]
    sem_g = sems[0:_NB]
    sem_e = sems[_NB:2 * _NB]
    sem_s = sems[2 * _NB:3 * _NB]

    c = lax.axis_index("c")
    s = lax.axis_index("s")
    w = s * _NC + c
    ebase = w * _EPW

    pltpu.sync_copy(packed_hbm.at[pl.ds(ebase, _EPW)], packed_all)

    def unpack(j, b):
        for g in range(_K // 16):
            v = packed_all[pl.ds(j * _K + g * 16, 16)]
            srcv[b][pl.ds(g * 16, 16)] = v & 0x3FFF
            dstv[b][pl.ds(g * 16, 16)] = lax.shift_right_logical(v, 14)

    def issue_loads(j, b):
        base = ebase + j * _K
        pltpu.async_copy(node_hbm.at[srcv[b]], rows[b], sem_g[b])
        pltpu.async_copy(edge_hbm.at[pl.ds(base, _K)], msg[b], sem_e[b])

    def wait_loads(j, b):
        base = ebase + j * _K
        pltpu.make_async_copy(node_hbm.at[srcv[b]], rows[b],
                              sem_g[b]).wait()
        pltpu.make_async_copy(edge_hbm.at[pl.ds(base, _K)], msg[b],
                              sem_e[b]).wait()

    def compute(b):
        m, x = msg[b], rows[b]

        def row2(r2, rc):
            for dr in range(2):
                r = r2 * 2 + dr
                for cc in range(_D // 16):
                    sl = pl.ds(cc * 16, 16)
                    m[r, sl] = jnp.maximum(m[r, sl] + x[r, sl], 0.0)
            return rc

        lax.fori_loop(0, _K // 2, row2, 0)

    def issue_scatter(b):
        pltpu.async_copy(msg[b], acc_sh.at[dstv[b]], sem_s[b], add=True)

    def wait_scatter(b):
        pltpu.make_async_copy(msg[b], acc_sh.at[dstv[b]], sem_s[b]).wait()

    def zrow(r, carry):
        for cc in range(_D // 16):
            rows[0][r, pl.ds(cc * 16, 16)] = jnp.zeros((16,), jnp.float32)
        return carry

    lax.fori_loop(0, _K, zrow, 0)

    def zchunk(i, carry):
        ch = s + i * _NS

        @pl.when(ch < _NZC)
        def _():
            off = jnp.minimum(ch * _K, _N - _K)
            pltpu.sync_copy(rows[0], acc_sh.at[pl.ds(off, _K)])

        return carry

    lax.fori_loop(0, _RCPT, zchunk, 0)
    unpack(0, 0)
    issue_loads(0, 0)
    plsc.subcore_barrier()

    def pair(t, carry):
        for sstep in range(2):
            j = 2 * t + sstep
            b = sstep
            o = 1 - sstep

            @pl.when(j >= 1)
            def _():
                wait_scatter(o)

            @pl.when(j + 1 < _NCH)
            def _():
                unpack(j + 1, o)
                issue_loads(j + 1, o)

            wait_loads(j, b)
            compute(b)
            issue_scatter(b)
        return carry

    lax.fori_loop(0, _NCH // 2, pair, 0)

    wait_scatter(1)
    v = packed_all[pl.ds(_NCH * _K, _TAIL)]
    srcv_t[...] = v & 0x3FFF
    dstv_t[...] = lax.shift_right_logical(v, 14)
    tbase = ebase + _NCH * _K
    pltpu.async_copy(node_hbm.at[srcv_t], rows[0].at[pl.ds(0, _TAIL)],
                     sem_g[0])
    pltpu.async_copy(edge_hbm.at[pl.ds(tbase, _TAIL)],
                     msg[0].at[pl.ds(0, _TAIL)], sem_e[0])
    pltpu.make_async_copy(node_hbm.at[srcv_t], rows[0].at[pl.ds(0, _TAIL)],
                          sem_g[0]).wait()
    pltpu.make_async_copy(edge_hbm.at[pl.ds(tbase, _TAIL)],
                          msg[0].at[pl.ds(0, _TAIL)], sem_e[0]).wait()

    def rowt(r, rc):
        for cc in range(_D // 16):
            sl = pl.ds(cc * 16, 16)
            msg[0][r, sl] = jnp.maximum(msg[0][r, sl] + rows[0][r, sl], 0.0)
        return rc

    lax.fori_loop(0, _TAIL, rowt, 0)
    pltpu.async_copy(msg[0].at[pl.ds(0, _TAIL)], acc_sh.at[dstv_t],
                     sem_s[0], add=True)
    pltpu.make_async_copy(msg[0].at[pl.ds(0, _TAIL)], acc_sh.at[dstv_t],
                          sem_s[0]).wait()
    plsc.subcore_barrier()

    def ochunk(i, carry):
        ch = s + i * _NS

        @pl.when(ch < _NZC)
        def _():
            off = jnp.minimum(ch * _K, _N - _K)
            pltpu.sync_copy(acc_sh.at[pl.ds(off, _K)], msg[0])
            pltpu.sync_copy(msg[0], out_hbm.at[c, pl.ds(off, _K)])

        return carry

    lax.fori_loop(0, _RCPT, ochunk, 0)


@functools.cache
def _sc_conv():
    return functools.partial(
        pl.kernel,
        out_type=jax.ShapeDtypeStruct((_NC, _N, _D), jnp.float32),
        mesh=plsc.VectorSubcoreMesh(core_axis_name="c", subcore_axis_name="s",
                                    num_cores=_NC, num_subcores=_NS),
        scratch_types=(
            [pltpu.VMEM((_EPW,), jnp.int32)]
            + [pltpu.VMEM((_K,), jnp.int32) for _ in range(2 * _NB)]
            + [pltpu.VMEM((_TAIL,), jnp.int32) for _ in range(2)]
            + [pltpu.VMEM((_K, _D), jnp.float32) for _ in range(2 * _NB)]
            + [pltpu.VMEM_SHARED((_N, _D), jnp.float32)]
            + [pltpu.SemaphoreType.DMA for _ in range(3 * _NB)]
        ),
    )(_sc_conv_body)


def _tc_body(node_ref, agg_ref, w1_ref, b1_ref, w2_ref, b2_ref,
             gamma_ref, beta_ref, out_ref):
    h = node_ref[...] + agg_ref[0] + agg_ref[1]
    h = jnp.maximum(
        lax.dot_general(h, w1_ref[...], (((1,), (0,)), ((), ())),
                        preferred_element_type=jnp.float32) + b1_ref[...], 0.0)
    h = lax.dot_general(h, w2_ref[...], (((1,), (0,)), ((), ())),
                        preferred_element_type=jnp.float32) + b2_ref[...]
    mean = jnp.mean(h, axis=0, keepdims=True)
    var = jnp.mean(jnp.square(h - mean), axis=0, keepdims=True)
    out_ref[...] = ((h - mean) * lax.rsqrt(var + 1e-5) * gamma_ref[...]
                    + beta_ref[...])


_tc_finish = pl.pallas_call(
    _tc_body,
    out_shape=jax.ShapeDtypeStruct((_N, _D), jnp.float32),
)


def kernel(node_feats, edge_feats, W1, b1, W2, b2, gamma, beta, edge_index):
    src = edge_index[0]
    dst = edge_index[1]
    packed = jnp.bitwise_or(src, jnp.left_shift(dst, 14))
    agg2 = _sc_conv()(node_feats, packed, edge_feats)
    return _tc_finish(node_feats, agg2,
                      W1, b1.reshape(1, _D),
                      W2, b2.reshape(1, _D),
                      gamma.reshape(1, _D), beta.reshape(1, _D))


# speedup vs baseline: 1.1305x; 1.1305x over previous
import functools

import jax
import jax.numpy as jnp
from jax import lax
from jax.experimental import pallas as pl
from jax.experimental.pallas import tpu as pltpu
from jax.experimental.pallas import tpu_sc as plsc

_N = 10000
_E = 320000
_D = 128
_NC = 2
_NS = 16
_NW = _NC * _NS
_EPW = _E // _NW
_K = 40
_NCH = _EPW // _K
_NSEG = 2
_SCH = _NCH // _NSEG
_NZC = _N // _K
_RCPT = -(-_NZC // _NS)
_NB = 4


def _sc_conv_body(node_hbm, packed_hbm, edge_hbm, out_hbm, *refs):
    packed_seg = refs[0]
    srcv = refs[1:1 + _NB]
    dstv = refs[1 + _NB:1 + 2 * _NB]
    rows = refs[1 + 2 * _NB:1 + 3 * _NB]
    msg = refs[1 + 3 * _NB:1 + 4 * _NB]
    acc_sh = refs[1 + 4 * _NB]
    sems = refs[2 + 4 * _NB:]
    sem_g = sems[0:_NB]
    sem_e = sems[_NB:2 * _NB]
    sem_s = sems[2 * _NB:3 * _NB]

    c = lax.axis_index("c")
    s = lax.axis_index("s")
    w = s * _NC + c
    ebase = w * _EPW

    def unpack(jl, b):
        for g in range(_K // 16):
            v = packed_seg[pl.ds(jl * _K + g * 16, 16)]
            srcv[b][pl.ds(g * 16, 16)] = v & 0x3FFF
            dstv[b][pl.ds(g * 16, 16)] = lax.shift_right_logical(v, 14)
        if _K % 16:
            off = _K - 16
            vv = packed_seg[pl.ds(jl * _K + off, 16)]
            srcv[b][pl.ds(off, 16)] = vv & 0x3FFF
            dstv[b][pl.ds(off, 16)] = lax.shift_right_logical(vv, 14)

    def make_seg(p):
        sbase = ebase + p * _SCH * _K

        def issue_loads(jl, b):
            base = sbase + jl * _K
            pltpu.async_copy(node_hbm.at[srcv[b]], rows[b], sem_g[b])
            pltpu.async_copy(edge_hbm.at[pl.ds(base, _K)], msg[b], sem_e[b])

        def wait_loads(jl, b):
            base = sbase + jl * _K
            pltpu.make_async_copy(node_hbm.at[srcv[b]], rows[b],
                                  sem_g[b]).wait()
            pltpu.make_async_copy(edge_hbm.at[pl.ds(base, _K)], msg[b],
                                  sem_e[b]).wait()

        return issue_loads, wait_loads

    def compute(b):
        m, x = msg[b], rows[b]

        def row2(r2, rc):
            for dr in range(2):
                r = r2 * 2 + dr
                for cc in range(_D // 16):
                    sl = pl.ds(cc * 16, 16)
                    m[r, sl] = jnp.maximum(m[r, sl] + x[r, sl], 0.0)
            return rc

        lax.fori_loop(0, _K // 2, row2, 0)

    def issue_scatter(b):
        pltpu.async_copy(msg[b], acc_sh.at[dstv[b]], sem_s[b], add=True)

    def wait_scatter(b):
        pltpu.make_async_copy(msg[b], acc_sh.at[dstv[b]], sem_s[b]).wait()

    def run_segment(p, zero_phase):
        issue_loads, wait_loads = make_seg(p)
        pltpu.sync_copy(
            packed_hbm.at[pl.ds(ebase + p * _SCH * _K, _SCH * _K)],
            packed_seg)
        unpack(0, 0)
        issue_loads(0, 0)
        unpack(1, 1)
        issue_loads(1, 1)

        if zero_phase:
            def zrow(r, carry):
                for cc in range(_D // 16):
                    rows[2][r, pl.ds(cc * 16, 16)] = jnp.zeros(
                        (16,), jnp.float32)
                return carry

            lax.fori_loop(0, _K, zrow, 0)

            def zchunk(i, carry):
                ch = s + i * _NS

                @pl.when(ch < _NZC)
                def _():
                    pltpu.sync_copy(rows[2], acc_sh.at[pl.ds(ch * _K, _K)])

                return carry

            lax.fori_loop(0, _RCPT, zchunk, 0)
            plsc.subcore_barrier()

        def quad(t, carry):
            for sstep in range(_NB):
                jl = _NB * t + sstep
                b = sstep
                b2 = (sstep + 2) % _NB

                @pl.when(jl >= 2)
                def _():
                    wait_scatter(b2)

                @pl.when(jl + 2 < _SCH)
                def _():
                    unpack(jl + 2, b2)
                    issue_loads(jl + 2, b2)

                wait_loads(jl, b)
                compute(b)
                issue_scatter(b)
            return carry

        lax.fori_loop(0, _SCH // _NB, quad, 0)
        wait_scatter(2)
        wait_loads(_SCH - 1, 0)
        compute(0)
        issue_scatter(0)
        wait_scatter(3)
        wait_scatter(0)

    run_segment(0, zero_phase=True)
    run_segment(1, zero_phase=False)
    plsc.subcore_barrier()

    def ochunk(i, carry):
        ch = s + i * _NS

        @pl.when(ch < _NZC)
        def _():
            pltpu.sync_copy(acc_sh.at[pl.ds(ch * _K, _K)], msg[0])
            pltpu.sync_copy(msg[0], out_hbm.at[c, pl.ds(ch * _K, _K)])

        return carry

    lax.fori_loop(0, _RCPT, ochunk, 0)


@functools.cache
def _sc_conv():
    return functools.partial(
        pl.kernel,
        out_type=jax.ShapeDtypeStruct((_NC, _N, _D), jnp.float32),
        mesh=plsc.VectorSubcoreMesh(core_axis_name="c", subcore_axis_name="s",
                                    num_cores=_NC, num_subcores=_NS),
        scratch_types=(
            [pltpu.VMEM((_SCH * _K,), jnp.int32)]
            + [pltpu.VMEM((_K,), jnp.int32) for _ in range(2 * _NB)]
            + [pltpu.VMEM((_K, _D), jnp.float32) for _ in range(2 * _NB)]
            + [pltpu.VMEM_SHARED((_N, _D), jnp.float32)]
            + [pltpu.SemaphoreType.DMA for _ in range(3 * _NB)]
        ),
    )(_sc_conv_body)


def _tc_body(node_ref, agg_ref, w1_ref, b1_ref, w2_ref, b2_ref,
             gamma_ref, beta_ref, out_ref):
    h = node_ref[...] + agg_ref[0] + agg_ref[1]
    h = jnp.maximum(
        lax.dot_general(h, w1_ref[...], (((1,), (0,)), ((), ())),
                        preferred_element_type=jnp.float32) + b1_ref[...], 0.0)
    h = lax.dot_general(h, w2_ref[...], (((1,), (0,)), ((), ())),
                        preferred_element_type=jnp.float32) + b2_ref[...]
    mean = jnp.mean(h, axis=0, keepdims=True)
    var = jnp.mean(jnp.square(h - mean), axis=0, keepdims=True)
    out_ref[...] = ((h - mean) * lax.rsqrt(var + 1e-5) * gamma_ref[...]
                    + beta_ref[...])


_tc_finish = pl.pallas_call(
    _tc_body,
    out_shape=jax.ShapeDtypeStruct((_N, _D), jnp.float32),
)


def kernel(node_feats, edge_feats, W1, b1, W2, b2, gamma, beta, edge_index):
    src = edge_index[0]
    dst = edge_index[1]
    packed = jnp.bitwise_or(src, jnp.left_shift(dst, 14))
    agg2 = _sc_conv()(node_feats, packed, edge_feats)
    return _tc_finish(node_feats, agg2,
                      W1, b1.reshape(1, _D),
                      W2, b2.reshape(1, _D),
                      gamma.reshape(1, _D), beta.reshape(1, _D))


# speedup vs baseline: 1.1930x; 1.0552x over previous
import functools

import jax
import jax.numpy as jnp
from jax import lax
from jax.experimental import pallas as pl
from jax.experimental.pallas import tpu as pltpu
from jax.experimental.pallas import tpu_sc as plsc

_N = 10000
_E = 320000
_D = 128
_NC = 2
_NS = 16
_NW = _NC * _NS
_EPW = _E // _NW
_K = 40
_NCH = _EPW // _K
_NRC = _N // _K
_RCPT = -(-_NRC // _NS)


_NB = 4
_NI = 8


def _sc_conv_body(node_hbm, src_hbm, dst_hbm, edge_hbm, out_hbm, *refs):
    srcv = refs[0:_NI]
    dstv = refs[_NI:2 * _NI]
    rows = refs[2 * _NI:2 * _NI + _NB]
    msg = refs[2 * _NI + _NB:2 * _NI + 2 * _NB]
    acc_sh = refs[2 * _NI + 2 * _NB]
    sems = refs[2 * _NI + 2 * _NB + 1:]
    sem_r = sems[0:_NI]
    sem_i = sems[_NI:2 * _NI]
    sem_g = sems[2 * _NI:2 * _NI + _NB]
    sem_e = sems[2 * _NI + _NB:2 * _NI + 2 * _NB]
    sem_s = sems[2 * _NI + 2 * _NB:2 * _NI + 3 * _NB]

    c = lax.axis_index("c")
    s = lax.axis_index("s")
    w = s * _NC + c
    ebase = w * _EPW

    def issue_idx(j, bi):
        base = ebase + j * _K
        pltpu.async_copy(src_hbm.at[pl.ds(base, _K)], srcv[bi], sem_r[bi])
        pltpu.async_copy(dst_hbm.at[pl.ds(base, _K)], dstv[bi], sem_i[bi])

    def wait_idx(j, bi):
        base = ebase + j * _K
        pltpu.make_async_copy(src_hbm.at[pl.ds(base, _K)], srcv[bi],
                              sem_r[bi]).wait()
        pltpu.make_async_copy(dst_hbm.at[pl.ds(base, _K)], dstv[bi],
                              sem_i[bi]).wait()

    def issue_loads(j, bi, b):
        base = ebase + j * _K
        pltpu.async_copy(node_hbm.at[srcv[bi]], rows[b], sem_g[b])
        pltpu.async_copy(edge_hbm.at[pl.ds(base, _K)], msg[b], sem_e[b])

    def wait_loads(j, bi, b):
        base = ebase + j * _K
        pltpu.make_async_copy(node_hbm.at[srcv[bi]], rows[b],
                              sem_g[b]).wait()
        pltpu.make_async_copy(edge_hbm.at[pl.ds(base, _K)], msg[b],
                              sem_e[b]).wait()

    def compute(b):
        m, x = msg[b], rows[b]

        def row2(r2, rc):
            for dr in range(2):
                r = r2 * 2 + dr
                for cc in range(_D // 16):
                    sl = pl.ds(cc * 16, 16)
                    m[r, sl] = jnp.maximum(m[r, sl] + x[r, sl], 0.0)
            return rc

        lax.fori_loop(0, _K // 2, row2, 0)

    def issue_scatter(bi, b):
        pltpu.async_copy(msg[b], acc_sh.at[dstv[bi]], sem_s[b], add=True)

    def wait_scatter(bi, b):
        pltpu.make_async_copy(msg[b], acc_sh.at[dstv[bi]], sem_s[b]).wait()

    issue_idx(0, 0)
    issue_idx(1, 1)
    issue_idx(2, 2)
    wait_idx(0, 0)
    issue_loads(0, 0, 0)
    wait_idx(1, 1)
    issue_loads(1, 1, 1)

    def zrow(r, carry):
        for cc in range(_D // 16):
            rows[2][r, pl.ds(cc * 16, 16)] = jnp.zeros((16,), jnp.float32)
        return carry

    lax.fori_loop(0, _K, zrow, 0)

    for zi in range(_RCPT):
        zch = s + zi * _NS

        @pl.when(zch < _NRC)
        def _():
            pltpu.async_copy(rows[2], acc_sh.at[pl.ds(zch * _K, _K)],
                             sem_s[zi % _NB])

    for zi in range(_RCPT):
        zch = s + zi * _NS

        @pl.when(zch < _NRC)
        def _():
            pltpu.make_async_copy(rows[2], acc_sh.at[pl.ds(zch * _K, _K)],
                                  sem_s[zi % _NB]).wait()

    plsc.subcore_barrier()

    def oct_(t, carry):
        for sstep in range(_NI):
            j = _NI * t + sstep
            b = sstep % _NB
            bi = sstep

            @pl.when(j + 3 < _NCH)
            def _():
                issue_idx(j + 3, (sstep + 3) % _NI)

            @pl.when(j >= 2)
            def _():
                wait_scatter((sstep - 2) % _NI, (sstep + 2) % _NB)

            @pl.when(j + 2 < _NCH)
            def _():
                wait_idx(j + 2, (sstep + 2) % _NI)
                issue_loads(j + 2, (sstep + 2) % _NI, (sstep + 2) % _NB)

            wait_loads(j, bi, b)
            compute(b)
            issue_scatter(bi, b)
        return carry

    lax.fori_loop(0, _NCH // _NI, oct_, 0)
    for i in range(_NCH % _NI):
        j = (_NCH // _NI) * _NI + i
        wait_scatter((i - 2) % _NI, (i + 2) % _NB)
        wait_loads(j, i, i % _NB)
        compute(i % _NB)
        issue_scatter(i, i % _NB)
    for i in range(_NCH % _NI):
        wait_scatter(i, i % _NB)
    plsc.subcore_barrier()

    def oread(i, b):
        ch = s + i * _NS
        pltpu.async_copy(acc_sh.at[pl.ds(ch * _K, _K)], msg[b], sem_g[b])

    def oread_wait(i, b):
        ch = s + i * _NS
        pltpu.make_async_copy(acc_sh.at[pl.ds(ch * _K, _K)], msg[b],
                              sem_g[b]).wait()

    def owrite(i, b):
        ch = s + i * _NS
        pltpu.async_copy(msg[b], out_hbm.at[c, pl.ds(ch * _K, _K)],
                         sem_e[b])

    def owrite_wait(i, b):
        ch = s + i * _NS
        pltpu.make_async_copy(msg[b], out_hbm.at[c, pl.ds(ch * _K, _K)],
                              sem_e[b]).wait()

    for oi in range(2):
        @pl.when(s + oi * _NS < _NRC)
        def _():
            oread(oi, oi)

    for oi in range(_RCPT):
        ob = oi % _NB

        if oi >= 2:
            @pl.when(s + (oi - 2) * _NS < _NRC)
            def _():
                owrite_wait(oi - 2, (oi - 2) % _NB)

        if oi + 2 < _RCPT:
            @pl.when(s + (oi + 2) * _NS < _NRC)
            def _():
                oread(oi + 2, (oi + 2) % _NB)

        @pl.when(s + oi * _NS < _NRC)
        def _():
            oread_wait(oi, ob)
            owrite(oi, ob)

    for oi in range(_RCPT - 2, _RCPT):
        @pl.when(s + oi * _NS < _NRC)
        def _():
            owrite_wait(oi, oi % _NB)


@functools.cache
def _sc_conv():
    return functools.partial(
        pl.kernel,
        out_type=jax.ShapeDtypeStruct((_NC, _N, _D), jnp.float32),
        mesh=plsc.VectorSubcoreMesh(core_axis_name="c", subcore_axis_name="s",
                                    num_cores=_NC, num_subcores=_NS),
        scratch_types=(
            [pltpu.VMEM((_K,), jnp.int32) for _ in range(2 * _NI)]
            + [pltpu.VMEM((_K, _D), jnp.float32) for _ in range(2 * _NB)]
            + [pltpu.VMEM_SHARED((_N, _D), jnp.float32)]
            + [pltpu.SemaphoreType.DMA for _ in range(2 * _NI + 3 * _NB)]
        ),
    )(_sc_conv_body)


def _tc_body(node_ref, agg_ref, w1_ref, b1_ref, w2_ref, b2_ref,
             gamma_ref, beta_ref, out_ref):
    h = node_ref[...] + agg_ref[0] + agg_ref[1]
    h = jnp.maximum(
        lax.dot_general(h, w1_ref[...], (((1,), (0,)), ((), ())),
                        preferred_element_type=jnp.float32) + b1_ref[...], 0.0)
    h = lax.dot_general(h, w2_ref[...], (((1,), (0,)), ((), ())),
                        preferred_element_type=jnp.float32) + b2_ref[...]
    mean = jnp.mean(h, axis=0, keepdims=True)
    var = jnp.mean(jnp.square(h - mean), axis=0, keepdims=True)
    out_ref[...] = ((h - mean) * lax.rsqrt(var + 1e-5) * gamma_ref[...]
                    + beta_ref[...])


_tc_finish = pl.pallas_call(
    _tc_body,
    out_shape=jax.ShapeDtypeStruct((_N, _D), jnp.float32),
)


def kernel(node_feats, edge_feats, W1, b1, W2, b2, gamma, beta, edge_index):
    src = edge_index[0]
    dst = edge_index[1]
    agg2 = _sc_conv()(node_feats, src, dst, edge_feats)
    return _tc_finish(node_feats, agg2,
                      W1, b1.reshape(1, _D),
                      W2, b2.reshape(1, _D),
                      gamma.reshape(1, _D), beta.reshape(1, _D))


# speedup vs baseline: 1.2047x; 1.0098x over previous
import functools

import jax
import jax.numpy as jnp
from jax import lax
from jax.experimental import pallas as pl
from jax.experimental.pallas import tpu as pltpu
from jax.experimental.pallas import tpu_sc as plsc

_N = 10000
_E = 320000
_D = 128
_NC = 2
_NS = 16
_NW = _NC * _NS
_EPW = _E // _NW
_K = 40
_NCH = _EPW // _K
_NRC = _N // _K
_RCPT = -(-_NRC // _NS)


_NB = 4
_NI = 8


def _sc_conv_body(node_hbm, src_hbm, dst_hbm, edge_hbm, out_hbm, *refs):
    srcv = refs[0:_NI]
    dstv = refs[_NI:2 * _NI]
    rows = refs[2 * _NI:2 * _NI + _NB]
    msg = refs[2 * _NI + _NB:2 * _NI + 2 * _NB]
    acc_sh = refs[2 * _NI + 2 * _NB]
    sems = refs[2 * _NI + 2 * _NB + 1:]
    sem_r = sems[0:_NI]
    sem_i = sems[_NI:2 * _NI]
    sem_g = sems[2 * _NI:2 * _NI + _NB]
    sem_e = sems[2 * _NI + _NB:2 * _NI + 2 * _NB]
    sem_s = sems[2 * _NI + 2 * _NB:2 * _NI + 3 * _NB]

    c = lax.axis_index("c")
    s = lax.axis_index("s")
    w = s * _NC + c
    ebase = w * _EPW

    def issue_idx(j, bi):
        base = ebase + j * _K
        pltpu.async_copy(src_hbm.at[pl.ds(base, _K)], srcv[bi], sem_r[bi])
        pltpu.async_copy(dst_hbm.at[pl.ds(base, _K)], dstv[bi], sem_i[bi])

    def wait_idx(j, bi):
        base = ebase + j * _K
        pltpu.make_async_copy(src_hbm.at[pl.ds(base, _K)], srcv[bi],
                              sem_r[bi]).wait()
        pltpu.make_async_copy(dst_hbm.at[pl.ds(base, _K)], dstv[bi],
                              sem_i[bi]).wait()

    def issue_loads(j, bi, b):
        base = ebase + j * _K
        pltpu.async_copy(node_hbm.at[srcv[bi]], rows[b], sem_g[b])
        pltpu.async_copy(edge_hbm.at[pl.ds(base, _K)], msg[b], sem_e[b])

    def wait_loads(j, bi, b):
        base = ebase + j * _K
        pltpu.make_async_copy(node_hbm.at[srcv[bi]], rows[b],
                              sem_g[b]).wait()
        pltpu.make_async_copy(edge_hbm.at[pl.ds(base, _K)], msg[b],
                              sem_e[b]).wait()

    def compute(b):
        m, x = msg[b], rows[b]

        def row2(r2, rc):
            for dr in range(2):
                r = r2 * 2 + dr
                for cc in range(_D // 16):
                    sl = pl.ds(cc * 16, 16)
                    m[r, sl] = jnp.maximum(m[r, sl] + x[r, sl], 0.0)
            return rc

        lax.fori_loop(0, _K // 2, row2, 0)

    def issue_scatter(bi, b):
        pltpu.async_copy(msg[b], acc_sh.at[dstv[bi]], sem_s[b], add=True)

    def wait_scatter(bi, b):
        pltpu.make_async_copy(msg[b], acc_sh.at[dstv[bi]], sem_s[b]).wait()

    for pbi in range(5):
        issue_idx(pbi, pbi)
    wait_idx(0, 0)
    issue_loads(0, 0, 0)
    wait_idx(1, 1)
    issue_loads(1, 1, 1)

    def zrow(r, carry):
        for cc in range(_D // 16):
            rows[2][r, pl.ds(cc * 16, 16)] = jnp.zeros((16,), jnp.float32)
        return carry

    lax.fori_loop(0, _K, zrow, 0)

    for zi in range(_RCPT):
        zch = s + zi * _NS

        @pl.when(zch < _NRC)
        def _():
            pltpu.async_copy(rows[2], acc_sh.at[pl.ds(zch * _K, _K)],
                             sem_s[zi % _NB])

    for zi in range(_RCPT):
        zch = s + zi * _NS

        @pl.when(zch < _NRC)
        def _():
            pltpu.make_async_copy(rows[2], acc_sh.at[pl.ds(zch * _K, _K)],
                                  sem_s[zi % _NB]).wait()

    plsc.subcore_barrier()

    def oct_(t, carry):
        for sstep in range(_NI):
            j = _NI * t + sstep
            b = sstep % _NB
            bi = sstep

            @pl.when(j + 5 < _NCH)
            def _():
                issue_idx(j + 5, (sstep + 5) % _NI)

            @pl.when(j >= 2)
            def _():
                wait_scatter((sstep - 2) % _NI, (sstep + 2) % _NB)

            @pl.when(j + 2 < _NCH)
            def _():
                wait_idx(j + 2, (sstep + 2) % _NI)
                issue_loads(j + 2, (sstep + 2) % _NI, (sstep + 2) % _NB)

            wait_loads(j, bi, b)
            compute(b)
            issue_scatter(bi, b)
        return carry

    lax.fori_loop(0, _NCH // _NI, oct_, 0)
    for i in range(_NCH % _NI):
        j = (_NCH // _NI) * _NI + i
        wait_scatter((i - 2) % _NI, (i + 2) % _NB)
        wait_loads(j, i, i % _NB)
        compute(i % _NB)
        issue_scatter(i, i % _NB)
    for i in range(_NCH % _NI):
        wait_scatter(i, i % _NB)
    plsc.subcore_barrier()

    def oread(i, b):
        ch = s + i * _NS
        pltpu.async_copy(acc_sh.at[pl.ds(ch * _K, _K)], msg[b], sem_g[b])

    def oread_wait(i, b):
        ch = s + i * _NS
        pltpu.make_async_copy(acc_sh.at[pl.ds(ch * _K, _K)], msg[b],
                              sem_g[b]).wait()

    def owrite(i, b):
        ch = s + i * _NS
        pltpu.async_copy(msg[b], out_hbm.at[c, pl.ds(ch * _K, _K)],
                         sem_e[b])

    def owrite_wait(i, b):
        ch = s + i * _NS
        pltpu.make_async_copy(msg[b], out_hbm.at[c, pl.ds(ch * _K, _K)],
                              sem_e[b]).wait()

    for oi in range(2):
        @pl.when(s + oi * _NS < _NRC)
        def _():
            oread(oi, oi)

    for oi in range(_RCPT):
        ob = oi % _NB

        if oi >= 2:
            @pl.when(s + (oi - 2) * _NS < _NRC)
            def _():
                owrite_wait(oi - 2, (oi - 2) % _NB)

        if oi + 2 < _RCPT:
            @pl.when(s + (oi + 2) * _NS < _NRC)
            def _():
                oread(oi + 2, (oi + 2) % _NB)

        @pl.when(s + oi * _NS < _NRC)
        def _():
            oread_wait(oi, ob)
            owrite(oi, ob)

    for oi in range(_RCPT - 2, _RCPT):
        @pl.when(s + oi * _NS < _NRC)
        def _():
            owrite_wait(oi, oi % _NB)


@functools.cache
def _sc_conv():
    return functools.partial(
        pl.kernel,
        out_type=jax.ShapeDtypeStruct((_NC, _N, _D), jnp.float32),
        mesh=plsc.VectorSubcoreMesh(core_axis_name="c", subcore_axis_name="s",
                                    num_cores=_NC, num_subcores=_NS),
        scratch_types=(
            [pltpu.VMEM((_K,), jnp.int32) for _ in range(2 * _NI)]
            + [pltpu.VMEM((_K, _D), jnp.float32) for _ in range(2 * _NB)]
            + [pltpu.VMEM_SHARED((_N, _D), jnp.float32)]
            + [pltpu.SemaphoreType.DMA for _ in range(2 * _NI + 3 * _NB)]
        ),
    )(_sc_conv_body)


def _tc_body(node_ref, agg_ref, w1_ref, b1_ref, w2_ref, b2_ref,
             gamma_ref, beta_ref, out_ref):
    h = node_ref[...] + agg_ref[0] + agg_ref[1]
    h = jnp.maximum(
        lax.dot_general(h, w1_ref[...], (((1,), (0,)), ((), ())),
                        preferred_element_type=jnp.float32) + b1_ref[...], 0.0)
    h = lax.dot_general(h, w2_ref[...], (((1,), (0,)), ((), ())),
                        preferred_element_type=jnp.float32) + b2_ref[...]
    mean = jnp.mean(h, axis=0, keepdims=True)
    var = jnp.mean(jnp.square(h - mean), axis=0, keepdims=True)
    out_ref[...] = ((h - mean) * lax.rsqrt(var + 1e-5) * gamma_ref[...]
                    + beta_ref[...])


_tc_finish = pl.pallas_call(
    _tc_body,
    out_shape=jax.ShapeDtypeStruct((_N, _D), jnp.float32),
)


def kernel(node_feats, edge_feats, W1, b1, W2, b2, gamma, beta, edge_index):
    src = edge_index[0]
    dst = edge_index[1]
    agg2 = _sc_conv()(node_feats, src, dst, edge_feats)
    return _tc_finish(node_feats, agg2,
                      W1, b1.reshape(1, _D),
                      W2, b2.reshape(1, _D),
                      gamma.reshape(1, _D), beta.reshape(1, _D))
